# baseline (device time: 118061 ns/iter reference)
import jax
import jax.numpy as jnp
from jax import lax
from jax.experimental import pallas as pl
from jax.experimental.pallas import tpu as pltpu

N_DEV = 32


def kernel(x, w_mat):
    m, _ = x.shape
    _, n = w_mat.shape
    ch = m // N_DEV

    def body(x_ref, w_ref, out_ref, acc_ref, gather_ref, send_sems, recv1, recv2):
        my = lax.axis_index("i")

        bar = pltpu.get_barrier_semaphore()
        for o in range(1, N_DEV):
            peer = lax.rem(my + o, N_DEV)
            pl.semaphore_signal(
                bar, inc=1, device_id=(peer,),
                device_id_type=pl.DeviceIdType.MESH,
            )
        pl.semaphore_wait(bar, N_DEV - 1)

        acc_ref[...] = jnp.dot(
            x_ref[...], w_ref[...], preferred_element_type=jnp.float32
        )

        p1 = []
        for o in range(1, N_DEV):
            peer = lax.rem(my + o, N_DEV)
            rdma = pltpu.make_async_remote_copy(
                src_ref=acc_ref.at[pl.ds(peer * ch, ch), :],
                dst_ref=gather_ref.at[o - 1],
                send_sem=send_sems.at[o - 1],
                recv_sem=recv1.at[o - 1],
                device_id=(peer,),
                device_id_type=pl.DeviceIdType.MESH,
            )
            rdma.start()
            p1.append(rdma)
        for rdma in p1:
            rdma.wait_recv()

        red = acc_ref[pl.ds(my * ch, ch), :] + jnp.sum(gather_ref[...], axis=0)
        out_ref[pl.ds(my * ch, ch), :] = jnp.maximum(red, 0.0)

        for rdma in p1:
            rdma.wait_send()

        p2 = []
        for o in range(1, N_DEV):
            peer = lax.rem(my + o, N_DEV)
            rdma = pltpu.make_async_remote_copy(
                src_ref=out_ref.at[pl.ds(my * ch, ch), :],
                dst_ref=out_ref.at[pl.ds(my * ch, ch), :],
                send_sem=send_sems.at[o - 1],
                recv_sem=recv2.at[o - 1],
                device_id=(peer,),
                device_id_type=pl.DeviceIdType.MESH,
            )
            rdma.start()
            p2.append(rdma)
        for rdma in p2:
            rdma.wait_recv()
        for rdma in p2:
            rdma.wait_send()

    return pl.pallas_call(
        body,
        out_shape=jax.ShapeDtypeStruct((m, n), jnp.float32),
        in_specs=[
            pl.BlockSpec(memory_space=pltpu.VMEM),
            pl.BlockSpec(memory_space=pltpu.VMEM),
        ],
        out_specs=pl.BlockSpec(memory_space=pltpu.VMEM),
        scratch_shapes=[
            pltpu.VMEM((m, n), jnp.float32),
            pltpu.VMEM((N_DEV - 1, ch, n), jnp.float32),
            pltpu.SemaphoreType.DMA((N_DEV - 1,)),
            pltpu.SemaphoreType.DMA((N_DEV - 1,)),
            pltpu.SemaphoreType.DMA((N_DEV - 1,)),
        ],
        compiler_params=pltpu.CompilerParams(collective_id=0),
    )(x, w_mat)


# device time: 84030 ns/iter; 1.4050x vs baseline; 1.4050x over previous
import jax
import jax.numpy as jnp
from jax import lax
from jax.experimental import pallas as pl
from jax.experimental.pallas import tpu as pltpu

N_DEV = 32
CUBE = 8
NCUBE = 4


def _logical_id(q, p):
    yy = q % 2
    zz = q // 2
    jm = p % 4
    zh = p // 4
    z = 2 * zz + zh
    j = 4 * yy + jm
    return 8 * z + j


def kernel(x, w_mat):
    m, _ = x.shape
    _, n = w_mat.shape
    r1 = m // CUBE
    r2 = r1 // NCUBE

    def body(x_ref, w_ref, out_ref, acc_ref, red_ref, g1_ref, g2_ref,
             send_sems, rs1, rs2, rs3, rs4):
        my = lax.axis_index("i")
        j = my % 8
        zplane = my // 8
        q = (j // 4) % 2 + 2 * (zplane // 2)
        p = j % 4 + 4 * (zplane % 2)

        bar = pltpu.get_barrier_semaphore()
        for o in range(1, CUBE):
            peer = _logical_id(q, (p + o) % CUBE)
            pl.semaphore_signal(bar, inc=1, device_id=(peer,),
                                device_id_type=pl.DeviceIdType.MESH)
        for o in range(1, NCUBE):
            peer = _logical_id((q + o) % NCUBE, p)
            pl.semaphore_signal(bar, inc=1, device_id=(peer,),
                                device_id_type=pl.DeviceIdType.MESH)
        pl.semaphore_wait(bar, CUBE - 1 + NCUBE - 1)

        acc_ref[...] = jnp.dot(
            x_ref[...], w_ref[...], preferred_element_type=jnp.float32
        )

        l1 = []
        for o in range(1, CUBE):
            pp = (p + o) % CUBE
            rdma = pltpu.make_async_remote_copy(
                src_ref=acc_ref.at[pl.ds(pp * r1, r1), :],
                dst_ref=g1_ref.at[o - 1],
                send_sem=send_sems.at[o - 1],
                recv_sem=rs1.at[o - 1],
                device_id=(_logical_id(q, pp),),
                device_id_type=pl.DeviceIdType.MESH,
            )
            rdma.start()
            l1.append(rdma)
        for rdma in l1:
            rdma.wait_recv()

        red_ref[...] = acc_ref[pl.ds(p * r1, r1), :] + jnp.sum(
            g1_ref[...], axis=0
        )
        for rdma in l1:
            rdma.wait_send()

        l2 = []
        for o in range(1, NCUBE):
            qq = (q + o) % NCUBE
            rdma = pltpu.make_async_remote_copy(
                src_ref=red_ref.at[pl.ds(qq * r2, r2), :],
                dst_ref=g2_ref.at[o - 1],
                send_sem=send_sems.at[o - 1],
                recv_sem=rs2.at[o - 1],
                device_id=(_logical_id(qq, p),),
                device_id_type=pl.DeviceIdType.MESH,
            )
            rdma.start()
            l2.append(rdma)
        for rdma in l2:
            rdma.wait_recv()

        final = red_ref[pl.ds(q * r2, r2), :] + jnp.sum(g2_ref[...], axis=0)
        my_row = p * r1 + q * r2
        out_ref[pl.ds(my_row, r2), :] = jnp.maximum(final, 0.0)
        for rdma in l2:
            rdma.wait_send()

        l2b = []
        for o in range(1, NCUBE):
            qq = (q + o) % NCUBE
            rdma = pltpu.make_async_remote_copy(
                src_ref=out_ref.at[pl.ds(my_row, r2), :],
                dst_ref=out_ref.at[pl.ds(my_row, r2), :],
                send_sem=send_sems.at[o - 1],
                recv_sem=rs3.at[o - 1],
                device_id=(_logical_id(qq, p),),
                device_id_type=pl.DeviceIdType.MESH,
            )
            rdma.start()
            l2b.append(rdma)
        for rdma in l2b:
            rdma.wait_recv()
        for rdma in l2b:
            rdma.wait_send()

        l1b = []
        for o in range(1, CUBE):
            pp = (p + o) % CUBE
            rdma = pltpu.make_async_remote_copy(
                src_ref=out_ref.at[pl.ds(p * r1, r1), :],
                dst_ref=out_ref.at[pl.ds(p * r1, r1), :],
                send_sem=send_sems.at[o - 1],
                recv_sem=rs4.at[o - 1],
                device_id=(_logical_id(q, pp),),
                device_id_type=pl.DeviceIdType.MESH,
            )
            rdma.start()
            l1b.append(rdma)
        for rdma in l1b:
            rdma.wait_recv()
        for rdma in l1b:
            rdma.wait_send()

    return pl.pallas_call(
        body,
        out_shape=jax.ShapeDtypeStruct((m, n), jnp.float32),
        in_specs=[
            pl.BlockSpec(memory_space=pltpu.VMEM),
            pl.BlockSpec(memory_space=pltpu.VMEM),
        ],
        out_specs=pl.BlockSpec(memory_space=pltpu.VMEM),
        scratch_shapes=[
            pltpu.VMEM((m, n), jnp.float32),
            pltpu.VMEM((r1, n), jnp.float32),
            pltpu.VMEM((CUBE - 1, r1, n), jnp.float32),
            pltpu.VMEM((NCUBE - 1, r2, n), jnp.float32),
            pltpu.SemaphoreType.DMA((CUBE - 1,)),
            pltpu.SemaphoreType.DMA((CUBE - 1,)),
            pltpu.SemaphoreType.DMA((NCUBE - 1,)),
            pltpu.SemaphoreType.DMA((NCUBE - 1,)),
            pltpu.SemaphoreType.DMA((CUBE - 1,)),
        ],
        compiler_params=pltpu.CompilerParams(collective_id=0),
    )(x, w_mat)


# device time: 70791 ns/iter; 1.6677x vs baseline; 1.1870x over previous
import jax
import jax.numpy as jnp
from jax import lax
from jax.experimental import pallas as pl
from jax.experimental.pallas import tpu as pltpu

N_DEV = 32
CUBE = 8
NCUBE = 4
NCHUNK = 4


def _logical_id(q, p):
    z = 2 * (q // 2) + p // 4
    j = 4 * (q % 2) + p % 4
    return 8 * z + j


def kernel(x, w_mat):
    m, _ = x.shape
    _, n = w_mat.shape
    r1 = m // CUBE
    r2 = r1 // NCUBE
    cw = n // NCHUNK

    def body(x_ref, w_ref, out_ref, acc_ref, red_ref, g1_ref, g2_ref,
             ss1, ss2, ss3, ss4, rs1, rs2, rs3, rs4):
        my = lax.axis_index("i")
        j = my % 8
        zplane = my // 8
        q = (j // 4) % 2 + 2 * (zplane // 2)
        p = j % 4 + 4 * (zplane % 2)
        my_row = p * r1 + q * r2

        cube_peers = [(p + o) % CUBE for o in range(1, CUBE)]
        group_peers = [(q + o) % NCUBE for o in range(1, NCUBE)]

        bar = pltpu.get_barrier_semaphore()
        for pp in cube_peers:
            pl.semaphore_signal(bar, inc=1, device_id=(_logical_id(q, pp),),
                                device_id_type=pl.DeviceIdType.MESH)
        for qq in group_peers:
            pl.semaphore_signal(bar, inc=1, device_id=(_logical_id(qq, p),),
                                device_id_type=pl.DeviceIdType.MESH)
        pl.semaphore_wait(bar, CUBE - 1 + NCUBE - 1)

        acc_ref[...] = jnp.dot(
            x_ref[...], w_ref[...], preferred_element_type=jnp.float32
        )

        def col(c):
            return pl.ds(c * cw, cw)

        l1 = [[None] * (CUBE - 1) for _ in range(NCHUNK)]
        for c in range(NCHUNK):
            for oi, pp in enumerate(cube_peers):
                rdma = pltpu.make_async_remote_copy(
                    src_ref=acc_ref.at[pl.ds(pp * r1, r1), col(c)],
                    dst_ref=g1_ref.at[oi, :, col(c)],
                    send_sem=ss1.at[c, oi],
                    recv_sem=rs1.at[c, oi],
                    device_id=(_logical_id(q, pp),),
                    device_id_type=pl.DeviceIdType.MESH,
                )
                rdma.start()
                l1[c][oi] = rdma

        l2 = [[None] * (NCUBE - 1) for _ in range(NCHUNK)]
        for c in range(NCHUNK):
            for rdma in l1[c]:
                rdma.wait_recv()
            red_ref[:, col(c)] = acc_ref[pl.ds(p * r1, r1), col(c)] + jnp.sum(
                g1_ref[:, :, col(c)], axis=0
            )
            for oi, qq in enumerate(group_peers):
                rdma = pltpu.make_async_remote_copy(
                    src_ref=red_ref.at[pl.ds(qq * r2, r2), col(c)],
                    dst_ref=g2_ref.at[oi, :, col(c)],
                    send_sem=ss2.at[c, oi],
                    recv_sem=rs2.at[c, oi],
                    device_id=(_logical_id(qq, p),),
                    device_id_type=pl.DeviceIdType.MESH,
                )
                rdma.start()
                l2[c][oi] = rdma

        l2b = [[None] * (NCUBE - 1) for _ in range(NCHUNK)]
        for c in range(NCHUNK):
            for rdma in l2[c]:
                rdma.wait_recv()
            final = red_ref[pl.ds(q * r2, r2), col(c)] + jnp.sum(
                g2_ref[:, :, col(c)], axis=0
            )
            out_ref[pl.ds(my_row, r2), col(c)] = jnp.maximum(final, 0.0)
            for oi, qq in enumerate(group_peers):
                rdma = pltpu.make_async_remote_copy(
                    src_ref=out_ref.at[pl.ds(my_row, r2), col(c)],
                    dst_ref=out_ref.at[pl.ds(my_row, r2), col(c)],
                    send_sem=ss3.at[c, oi],
                    recv_sem=rs3.at[c, oi],
                    device_id=(_logical_id(qq, p),),
                    device_id_type=pl.DeviceIdType.MESH,
                )
                rdma.start()
                l2b[c][oi] = rdma

        l1b = [[None] * (CUBE - 1) for _ in range(NCHUNK)]
        for c in range(NCHUNK):
            for rdma in l2b[c]:
                rdma.wait_recv()
            for oi, pp in enumerate(cube_peers):
                rdma = pltpu.make_async_remote_copy(
                    src_ref=out_ref.at[pl.ds(p * r1, r1), col(c)],
                    dst_ref=out_ref.at[pl.ds(p * r1, r1), col(c)],
                    send_sem=ss4.at[c, oi],
                    recv_sem=rs4.at[c, oi],
                    device_id=(_logical_id(q, pp),),
                    device_id_type=pl.DeviceIdType.MESH,
                )
                rdma.start()
                l1b[c][oi] = rdma

        for c in range(NCHUNK):
            for rdma in l1b[c]:
                rdma.wait_recv()
        for group in (l1, l2, l2b, l1b):
            for c in range(NCHUNK):
                for rdma in group[c]:
                    rdma.wait_send()

    return pl.pallas_call(
        body,
        out_shape=jax.ShapeDtypeStruct((m, n), jnp.float32),
        in_specs=[
            pl.BlockSpec(memory_space=pltpu.VMEM),
            pl.BlockSpec(memory_space=pltpu.VMEM),
        ],
        out_specs=pl.BlockSpec(memory_space=pltpu.VMEM),
        scratch_shapes=[
            pltpu.VMEM((m, n), jnp.float32),
            pltpu.VMEM((r1, n), jnp.float32),
            pltpu.VMEM((CUBE - 1, r1, n), jnp.float32),
            pltpu.VMEM((NCUBE - 1, r2, n), jnp.float32),
            pltpu.SemaphoreType.DMA((NCHUNK, CUBE - 1)),
            pltpu.SemaphoreType.DMA((NCHUNK, NCUBE - 1)),
            pltpu.SemaphoreType.DMA((NCHUNK, NCUBE - 1)),
            pltpu.SemaphoreType.DMA((NCHUNK, CUBE - 1)),
            pltpu.SemaphoreType.DMA((NCHUNK, CUBE - 1)),
            pltpu.SemaphoreType.DMA((NCHUNK, NCUBE - 1)),
            pltpu.SemaphoreType.DMA((NCHUNK, NCUBE - 1)),
            pltpu.SemaphoreType.DMA((NCHUNK, CUBE - 1)),
        ],
        compiler_params=pltpu.CompilerParams(collective_id=0),
    )(x, w_mat)
